# Initial kernel scaffold; baseline (speedup 1.0000x reference)
import jax
import jax.numpy as jnp
from jax.experimental import pallas as pl


def kernel(edge_type, op_embedding):
    raise NotImplementedError("write your pallas kernel here")


# speedup vs baseline: 3.6053x; 3.6053x over previous
import functools

import jax
import jax.numpy as jnp
from jax import lax
from jax.experimental import pallas as pl
from jax.experimental.pallas import tpu as pltpu
from jax.experimental.pallas import tpu_sc as plsc

D_MODEL = 64
PAIR_W = 2 * D_MODEL
CHUNK = 64
NBUF = 8
LAG = 2
N_WORKERS = 32
N_CORES = 2


def _emb_kernel(n_pairs):
    n_chunks = n_pairs // (N_WORKERS * CHUNK)
    assert n_chunks % NBUF == 0 and n_chunks >= 2 * NBUF
    mesh = plsc.VectorSubcoreMesh(core_axis_name="c", subcore_axis_name="s")

    @functools.partial(
        pl.kernel,
        mesh=mesh,
        out_type=jax.ShapeDtypeStruct((n_pairs, PAIR_W), jnp.float32),
        scratch_types=[
            pltpu.VMEM((n_chunks, CHUNK), jnp.int32),
            pltpu.VMEM((NBUF, CHUNK, PAIR_W), jnp.float32),
            pltpu.SemaphoreType.DMA((NBUF,)),
            pltpu.SemaphoreType.DMA((NBUF,)),
        ],
    )
    def emb(idx_hbm, table_hbm, out_hbm, idx_v, rows_v, gsem, osem):
        wid = lax.axis_index("s") * N_CORES + lax.axis_index("c")
        chunk_base = wid * n_chunks

        pltpu.sync_copy(idx_hbm.at[pl.ds(chunk_base, n_chunks)], idx_v)

        def start_gather(j, b):
            pltpu.async_copy(table_hbm.at[idx_v.at[j]], rows_v.at[b], gsem.at[b])

        def wait_gather(j, b):
            pltpu.make_async_copy(
                table_hbm.at[idx_v.at[j]], rows_v.at[b], gsem.at[b]
            ).wait()

        def start_out(j, b):
            off = (chunk_base + j) * CHUNK
            pltpu.async_copy(rows_v.at[b], out_hbm.at[pl.ds(off, CHUNK)], osem.at[b])

        def wait_out(j, b):
            off = (chunk_base + j) * CHUNK
            pltpu.make_async_copy(
                rows_v.at[b], out_hbm.at[pl.ds(off, CHUNK)], osem.at[b]
            ).wait()

        for b in range(NBUF):
            start_gather(b, b)

        def group(g, carry):
            jo = g * NBUF
            for b in range(NBUF):
                j = jo + b
                bw = (b - LAG) % NBUF

                @pl.when(j >= LAG)
                def _():
                    wait_gather(j - LAG, bw)
                    start_out(j - LAG, bw)

                jg = j + 1
                bg = (b + 1) % NBUF

                @pl.when(jnp.logical_and(jg >= NBUF, jg < n_chunks))
                def _():
                    wait_out(jg - NBUF, bg)
                    start_gather(jg, bg)

            return carry

        lax.fori_loop(0, n_chunks // NBUF, group, 0)

        for j in range(n_chunks - LAG, n_chunks):
            wait_gather(j, j % NBUF)
            start_out(j, j % NBUF)
        for j in range(n_chunks - NBUF, n_chunks):
            wait_out(j, j % NBUF)

    return emb


def kernel(edge_type, op_embedding):
    b0, b1 = edge_type.shape
    n_rows = b0 * b1
    n_pairs = n_rows // 2
    v = op_embedding.shape[0]
    flat = edge_type.reshape(-1).astype(jnp.int32)
    pair_idx = (flat[0::2] * v + flat[1::2]).reshape(-1, CHUNK)
    table = op_embedding.astype(jnp.float32)
    ptable = jnp.concatenate(
        [
            jnp.broadcast_to(table[:, None, :], (v, v, D_MODEL)),
            jnp.broadcast_to(table[None, :, :], (v, v, D_MODEL)),
        ],
        axis=-1,
    ).reshape(v * v, PAIR_W)
    out = _emb_kernel(n_pairs)(pair_idx, ptable)
    return out.reshape(b0, b1, D_MODEL)


# speedup vs baseline: 4.0909x; 1.1347x over previous
import functools

import jax
import jax.numpy as jnp
from jax import lax
from jax.experimental import pallas as pl
from jax.experimental.pallas import tpu as pltpu
from jax.experimental.pallas import tpu_sc as plsc

D_MODEL = 64
PAIR_W = 2 * D_MODEL
CHUNK = 64
NBUF = 8
LAG = 2
N_WORKERS = 32
N_CORES = 2


def _emb_kernel(n_pairs):
    n_chunks = n_pairs // (N_WORKERS * CHUNK)
    assert n_chunks % NBUF == 0 and n_chunks >= 2 * NBUF
    mesh = plsc.VectorSubcoreMesh(core_axis_name="c", subcore_axis_name="s")

    @functools.partial(
        pl.kernel,
        mesh=mesh,
        out_type=jax.ShapeDtypeStruct((n_pairs, PAIR_W), jnp.float32),
        scratch_types=[
            pltpu.VMEM((n_chunks, CHUNK), jnp.int32),
            pltpu.VMEM((NBUF, CHUNK, PAIR_W), jnp.float32),
            pltpu.VMEM_SHARED((1600, PAIR_W), jnp.float32),
            pltpu.SemaphoreType.DMA((NBUF,)),
            pltpu.SemaphoreType.DMA((NBUF,)),
        ],
    )
    def emb(idx_hbm, table_hbm, out_hbm, idx_v, rows_v, table_sh, gsem, osem):
        wid = lax.axis_index("s") * N_CORES + lax.axis_index("c")
        chunk_base = wid * n_chunks

        @pl.when(lax.axis_index("s") == 0)
        def _():
            pltpu.sync_copy(table_hbm, table_sh)

        pltpu.sync_copy(idx_hbm.at[pl.ds(chunk_base, n_chunks)], idx_v)
        plsc.subcore_barrier()

        def start_gather(j, b):
            pltpu.async_copy(table_sh.at[idx_v.at[j]], rows_v.at[b], gsem.at[b])

        def wait_gather(j, b):
            pltpu.make_async_copy(
                table_sh.at[idx_v.at[j]], rows_v.at[b], gsem.at[b]
            ).wait()

        def start_out(j, b):
            off = (chunk_base + j) * CHUNK
            pltpu.async_copy(rows_v.at[b], out_hbm.at[pl.ds(off, CHUNK)], osem.at[b])

        def wait_out(j, b):
            off = (chunk_base + j) * CHUNK
            pltpu.make_async_copy(
                rows_v.at[b], out_hbm.at[pl.ds(off, CHUNK)], osem.at[b]
            ).wait()

        for b in range(NBUF):
            start_gather(b, b)

        def group(g, carry):
            jo = g * NBUF
            for b in range(NBUF):
                j = jo + b
                bw = (b - LAG) % NBUF

                @pl.when(j >= LAG)
                def _():
                    wait_gather(j - LAG, bw)
                    start_out(j - LAG, bw)

                jg = j + 1
                bg = (b + 1) % NBUF

                @pl.when(jnp.logical_and(jg >= NBUF, jg < n_chunks))
                def _():
                    wait_out(jg - NBUF, bg)
                    start_gather(jg, bg)

            return carry

        lax.fori_loop(0, n_chunks // NBUF, group, 0)

        for j in range(n_chunks - LAG, n_chunks):
            wait_gather(j, j % NBUF)
            start_out(j, j % NBUF)
        for j in range(n_chunks - NBUF, n_chunks):
            wait_out(j, j % NBUF)

    return emb


def kernel(edge_type, op_embedding):
    b0, b1 = edge_type.shape
    n_rows = b0 * b1
    n_pairs = n_rows // 2
    v = op_embedding.shape[0]
    flat = edge_type.reshape(-1).astype(jnp.int32)
    pair_idx = (flat[0::2] * v + flat[1::2]).reshape(-1, CHUNK)
    table = op_embedding.astype(jnp.float32)
    ptable = jnp.concatenate(
        [
            jnp.broadcast_to(table[:, None, :], (v, v, D_MODEL)),
            jnp.broadcast_to(table[None, :, :], (v, v, D_MODEL)),
        ],
        axis=-1,
    ).reshape(v * v, PAIR_W)
    out = _emb_kernel(n_pairs)(pair_idx, ptable)
    return out.reshape(b0, b1, D_MODEL)


# speedup vs baseline: 4.0936x; 1.0007x over previous
import functools

import jax
import jax.numpy as jnp
from jax import lax
from jax.experimental import pallas as pl
from jax.experimental.pallas import tpu as pltpu
from jax.experimental.pallas import tpu_sc as plsc

D_MODEL = 64
PAIR_W = 2 * D_MODEL
CHUNK = 128
NBUF = 4
LAG = 1
N_WORKERS = 32
N_CORES = 2


def _emb_kernel(n_pairs):
    n_chunks = n_pairs // (N_WORKERS * CHUNK)
    assert n_chunks % NBUF == 0 and n_chunks >= 2 * NBUF
    mesh = plsc.VectorSubcoreMesh(core_axis_name="c", subcore_axis_name="s")

    @functools.partial(
        pl.kernel,
        mesh=mesh,
        out_type=jax.ShapeDtypeStruct((n_pairs, PAIR_W), jnp.float32),
        scratch_types=[
            pltpu.VMEM((1, n_chunks, CHUNK), jnp.int32),
            pltpu.VMEM((NBUF, CHUNK, PAIR_W), jnp.float32),
            pltpu.VMEM_SHARED((1600, PAIR_W), jnp.float32),
            pltpu.SemaphoreType.DMA((NBUF,)),
            pltpu.SemaphoreType.DMA((NBUF,)),
        ],
    )
    def emb(idx_hbm, table_hbm, out_hbm, idx_v, rows_v, table_sh, gsem, osem):
        wid = lax.axis_index("s") * N_CORES + lax.axis_index("c")
        chunk_base = wid * n_chunks

        @pl.when(lax.axis_index("s") == 0)
        def _():
            pltpu.sync_copy(table_hbm, table_sh)

        pltpu.sync_copy(idx_hbm.at[pl.ds(wid, 1)], idx_v)
        plsc.subcore_barrier()

        def start_gather(j, b):
            pltpu.async_copy(table_sh.at[idx_v.at[0, j]], rows_v.at[b], gsem.at[b])

        def wait_gather(j, b):
            pltpu.make_async_copy(
                table_sh.at[idx_v.at[0, j]], rows_v.at[b], gsem.at[b]
            ).wait()

        def start_out(j, b):
            off = (chunk_base + j) * CHUNK
            pltpu.async_copy(rows_v.at[b], out_hbm.at[pl.ds(off, CHUNK)], osem.at[b])

        def wait_out(j, b):
            off = (chunk_base + j) * CHUNK
            pltpu.make_async_copy(
                rows_v.at[b], out_hbm.at[pl.ds(off, CHUNK)], osem.at[b]
            ).wait()

        for b in range(NBUF):
            start_gather(b, b)

        def group(g, carry):
            jo = g * NBUF
            for b in range(NBUF):
                j = jo + b
                bw = (b - LAG) % NBUF

                @pl.when(j >= LAG)
                def _():
                    wait_gather(j - LAG, bw)
                    start_out(j - LAG, bw)

                jg = j + 1
                bg = (b + 1) % NBUF

                @pl.when(jnp.logical_and(jg >= NBUF, jg < n_chunks))
                def _():
                    wait_out(jg - NBUF, bg)
                    start_gather(jg, bg)

            return carry

        lax.fori_loop(0, n_chunks // NBUF, group, 0)

        for j in range(n_chunks - LAG, n_chunks):
            wait_gather(j, j % NBUF)
            start_out(j, j % NBUF)
        for j in range(n_chunks - NBUF, n_chunks):
            wait_out(j, j % NBUF)

    return emb


def kernel(edge_type, op_embedding):
    b0, b1 = edge_type.shape
    n_rows = b0 * b1
    n_pairs = n_rows // 2
    v = op_embedding.shape[0]
    flat = edge_type.reshape(-1).astype(jnp.int32)
    pair_idx = (flat[0::2] * v + flat[1::2]).reshape(N_WORKERS, -1, CHUNK)
    table = op_embedding.astype(jnp.float32)
    ptable = jnp.concatenate(
        [
            jnp.broadcast_to(table[:, None, :], (v, v, D_MODEL)),
            jnp.broadcast_to(table[None, :, :], (v, v, D_MODEL)),
        ],
        axis=-1,
    ).reshape(v * v, PAIR_W)
    out = _emb_kernel(n_pairs)(pair_idx, ptable)
    return out.reshape(b0, b1, D_MODEL)


# speedup vs baseline: 4.0962x; 1.0006x over previous
import functools

import jax
import jax.numpy as jnp
from jax import lax
from jax.experimental import pallas as pl
from jax.experimental.pallas import tpu as pltpu
from jax.experimental.pallas import tpu_sc as plsc

D_MODEL = 64
PAIR_W = 2 * D_MODEL
CHUNK = 128
NBUF = 4
LAG = 1
N_WORKERS = 32
N_CORES = 2


def _emb_kernel(n_pairs):
    n_chunks = n_pairs // (N_WORKERS * CHUNK)
    assert n_chunks % NBUF == 0 and n_chunks >= 2 * NBUF
    mesh = plsc.VectorSubcoreMesh(core_axis_name="c", subcore_axis_name="s")

    @functools.partial(
        pl.kernel,
        mesh=mesh,
        out_type=jax.ShapeDtypeStruct((n_pairs, PAIR_W), jnp.float32),
        scratch_types=[
            pltpu.VMEM((1, n_chunks, CHUNK), jnp.int32),
            pltpu.VMEM((NBUF, CHUNK, PAIR_W), jnp.float32),
            pltpu.VMEM_SHARED((1600, PAIR_W), jnp.float32),
            pltpu.SemaphoreType.DMA((NBUF,)),
            pltpu.SemaphoreType.DMA((NBUF,)),
        ],
    )
    def emb(idx_hbm, table_hbm, out_hbm, idx_v, rows_v, table_sh, gsem, osem):
        wid = lax.axis_index("s") * N_CORES + lax.axis_index("c")
        chunk_base = wid * n_chunks

        @pl.when(lax.axis_index("s") == 0)
        def _():
            pltpu.sync_copy(table_hbm, table_sh)

        pltpu.sync_copy(idx_hbm.at[pl.ds(wid, 1)], idx_v)
        plsc.subcore_barrier()

        def start_gather(j, b):
            pltpu.async_copy(table_sh.at[pl.ds(0, CHUNK)], rows_v.at[b], gsem.at[b])

        def wait_gather(j, b):
            pltpu.make_async_copy(
                table_sh.at[pl.ds(0, CHUNK)], rows_v.at[b], gsem.at[b]
            ).wait()

        def start_out(j, b):
            off = (chunk_base + j) * CHUNK
            pltpu.async_copy(rows_v.at[b], out_hbm.at[pl.ds(off, CHUNK)], osem.at[b])

        def wait_out(j, b):
            off = (chunk_base + j) * CHUNK
            pltpu.make_async_copy(
                rows_v.at[b], out_hbm.at[pl.ds(off, CHUNK)], osem.at[b]
            ).wait()

        for b in range(NBUF):
            start_gather(b, b)

        def group(g, carry):
            jo = g * NBUF
            for b in range(NBUF):
                j = jo + b
                bw = (b - LAG) % NBUF

                @pl.when(j >= LAG)
                def _():
                    wait_gather(j - LAG, bw)
                    start_out(j - LAG, bw)

                jg = j + 1
                bg = (b + 1) % NBUF

                @pl.when(jnp.logical_and(jg >= NBUF, jg < n_chunks))
                def _():
                    wait_out(jg - NBUF, bg)
                    start_gather(jg, bg)

            return carry

        lax.fori_loop(0, n_chunks // NBUF, group, 0)

        for j in range(n_chunks - LAG, n_chunks):
            wait_gather(j, j % NBUF)
            start_out(j, j % NBUF)
        for j in range(n_chunks - NBUF, n_chunks):
            wait_out(j, j % NBUF)

    return emb


def kernel(edge_type, op_embedding):
    b0, b1 = edge_type.shape
    n_rows = b0 * b1
    n_pairs = n_rows // 2
    v = op_embedding.shape[0]
    flat = edge_type.reshape(-1).astype(jnp.int32)
    pair_idx = (flat[0::2] * v + flat[1::2]).reshape(N_WORKERS, -1, CHUNK)
    table = op_embedding.astype(jnp.float32)
    ptable = jnp.concatenate(
        [
            jnp.broadcast_to(table[:, None, :], (v, v, D_MODEL)),
            jnp.broadcast_to(table[None, :, :], (v, v, D_MODEL)),
        ],
        axis=-1,
    ).reshape(v * v, PAIR_W)
    out = _emb_kernel(n_pairs)(pair_idx, ptable)
    return out.reshape(b0, b1, D_MODEL)


# speedup vs baseline: 8.9315x; 2.1804x over previous
import functools

import jax
import jax.numpy as jnp
from jax import lax
from jax.experimental import pallas as pl
from jax.experimental.pallas import tpu as pltpu
from jax.experimental.pallas import tpu_sc as plsc

D_MODEL = 64
PAD_W = 128
CHUNK = 128
NBUF = 4
LAG = 1
N_WORKERS = 32
N_CORES = 2


def _emb_kernel(n_rows, n_table):
    n_chunks = n_rows // (N_WORKERS * CHUNK)
    assert n_chunks % NBUF == 0 and n_chunks >= 2 * NBUF
    mesh = plsc.VectorSubcoreMesh(core_axis_name="c", subcore_axis_name="s")

    @functools.partial(
        pl.kernel,
        mesh=mesh,
        out_type=jax.ShapeDtypeStruct((n_rows, D_MODEL), jnp.float32),
        scratch_types=[
            pltpu.VMEM((1, n_chunks, CHUNK), jnp.int32),
            pltpu.VMEM((NBUF, CHUNK, D_MODEL), jnp.float32),
            pltpu.VMEM_SHARED((n_table, D_MODEL), jnp.float32),
            pltpu.SemaphoreType.DMA((NBUF,)),
            pltpu.SemaphoreType.DMA((NBUF,)),
        ],
    )
    def emb(idx_hbm, table_hbm, out_hbm, idx_v, rows_v, table_sh, gsem, osem):
        wid = lax.axis_index("s") * N_CORES + lax.axis_index("c")
        chunk_base = wid * n_chunks

        @pl.when(lax.axis_index("s") == 0)
        def _():
            pltpu.sync_copy(table_hbm, table_sh)

        pltpu.sync_copy(idx_hbm.at[pl.ds(wid, 1)], idx_v)
        plsc.subcore_barrier()

        def start_gather(j, b):
            pltpu.async_copy(table_sh.at[idx_v.at[0, j]], rows_v.at[b], gsem.at[b])

        def wait_gather(j, b):
            pltpu.make_async_copy(
                table_sh.at[idx_v.at[0, j]], rows_v.at[b], gsem.at[b]
            ).wait()

        def start_out(j, b):
            off = (chunk_base + j) * CHUNK
            pltpu.async_copy(
                rows_v.at[b], out_hbm.at[pl.ds(off, CHUNK)], osem.at[b]
            )

        def wait_out(j, b):
            off = (chunk_base + j) * CHUNK
            pltpu.make_async_copy(
                rows_v.at[b], out_hbm.at[pl.ds(off, CHUNK)], osem.at[b]
            ).wait()

        for b in range(NBUF):
            start_gather(b, b)

        def group(g, carry):
            jo = g * NBUF
            for b in range(NBUF):
                j = jo + b
                bw = (b - LAG) % NBUF

                @pl.when(j >= LAG)
                def _():
                    wait_gather(j - LAG, bw)
                    start_out(j - LAG, bw)

                jg = j + 1
                bg = (b + 1) % NBUF

                @pl.when(jnp.logical_and(jg >= NBUF, jg < n_chunks))
                def _():
                    wait_out(jg - NBUF, bg)
                    start_gather(jg, bg)

            return carry

        lax.fori_loop(0, n_chunks // NBUF, group, 0)

        for j in range(n_chunks - LAG, n_chunks):
            wait_gather(j, j % NBUF)
            start_out(j, j % NBUF)
        for j in range(n_chunks - NBUF, n_chunks):
            wait_out(j, j % NBUF)

    return emb


def kernel(edge_type, op_embedding):
    b0, b1 = edge_type.shape
    n_rows = b0 * b1
    v = op_embedding.shape[0]
    idx = edge_type.reshape(N_WORKERS, -1, CHUNK).astype(jnp.int32)
    out = _emb_kernel(n_rows, v)(idx, op_embedding.astype(jnp.float32))
    return out.reshape(b0, b1, D_MODEL)
